# baseline (speedup 1.0000x reference)
import jax
import jax.numpy as jnp
from jax import lax
from jax.experimental import pallas as pl
from jax.experimental.pallas import tpu as pltpu
from jax.experimental.pallas import tpu_sc as plsc

USER_N = 50000
ITEM_N = 50000
NN = USER_N + ITEM_N
D = 32
HALF = NN // 2

NC = 2
NS = 16

LANE = 128
CHUNK_ROWS = 6
ROWS_PER_TILE = 792
N_CHUNKS = ROWS_PER_TILE // CHUNK_ROWS
R_PAD = ROWS_PER_TILE * NS
E_PAD = R_PAD * LANE
CHUNK_E = CHUNK_ROWS * LANE
N_GROUPS = CHUNK_E // 16

ZCH = 224
ACC_ROWS = NS * 14 * ZCH
TRASH = HALF
READ_ROWS = 3120
READ_TAIL = HALF - NS * READ_ROWS

_mesh = plsc.VectorSubcoreMesh(core_axis_name="c", subcore_axis_name="s",
                               num_cores=NC, num_subcores=NS)


def _layer_body(ego, srcr, dstr, wr, out, acc,
                src0, src1, dst0, dst1, w0, w1, rows0,
                esem, gsem, ssem):
    c = lax.axis_index("c")
    s = lax.axis_index("s")
    zero16 = jnp.zeros((16,), jnp.float32)
    half_base = c * HALF

    srcb = (src0, src1)
    dstb = (dst0, dst1)
    wb = (w0, w1)
    rows = rows0

    @pl.loop(0, ZCH)
    def _(r):
        rows0[r, pl.ds(0, 16)] = zero16
        rows0[r, pl.ds(16, 16)] = zero16

    for q in range(14):
        pltpu.sync_copy(rows0.at[pl.ds(0, ZCH)],
                        acc.at[pl.ds((s * 14 + q) * ZCH, ZCH)])
    plsc.subcore_barrier()

    def row0_of(q):
        return s * ROWS_PER_TILE + q * CHUNK_ROWS

    def fire_edges(q, b):
        r0 = row0_of(q)
        pltpu.async_copy(srcr.at[pl.ds(r0, CHUNK_ROWS)], srcb[b], esem)
        pltpu.async_copy(dstr.at[pl.ds(r0, CHUNK_ROWS)], dstb[b], esem)
        pltpu.async_copy(wr.at[pl.ds(r0, CHUNK_ROWS)], wb[b], esem)

    def wait_edges(b):
        pltpu.make_async_copy(srcr.at[pl.ds(0, CHUNK_ROWS)], srcb[b], esem).wait()
        pltpu.make_async_copy(dstr.at[pl.ds(0, CHUNK_ROWS)], dstb[b], esem).wait()
        pltpu.make_async_copy(wr.at[pl.ds(0, CHUNK_ROWS)], wb[b], esem).wait()

    def gathers(b, r_lo, r_hi):
        return [pltpu.async_copy(ego.at[srcb[b].at[r]],
                                 rows.at[pl.ds(r * LANE, LANE)], gsem)
                for r in range(r_lo, r_hi)]

    def scatters(b, r_lo, r_hi):
        return [pltpu.async_copy(rows.at[pl.ds(r * LANE, LANE)],
                                 acc.at[dstb[b].at[r]], ssem, add=True)
                for r in range(r_lo, r_hi)]

    def adj_dst(b):
        d_ref = dstb[b]
        for j in range(CHUNK_ROWS):
            for i in range(LANE // 16):
                dv = d_ref[j, pl.ds(i * 16, 16)]
                lv = dv - half_base
                inr = (lv >= 0) & (lv < HALF)
                d_ref[j, pl.ds(i * 16, 16)] = jnp.where(inr, lv, TRASH)

    def multiply(b, g_lo, g_hi):
        w_v = wb[b]

        @plsc.parallel_loop(g_lo, g_hi, 1)
        def _(g):
            j = g >> 3
            i = (g & 7) * 16
            w16 = w_v[j, pl.ds(i, 16)]
            e0 = g * 16
            for base in range(0, 16, 8):
                vals = []
                for l in range(base, base + 8):
                    a = rows[e0 + l, pl.ds(0, 16)]
                    bb = rows[e0 + l, pl.ds(16, 16)]
                    vals.append((l, a, bb, w16[l]))
                for l, a, bb, w in vals:
                    rows[e0 + l, pl.ds(0, 16)] = a * w
                    rows[e0 + l, pl.ds(16, 16)] = bb * w

    def body(q, b):
        nb = 1 - b
        HR = CHUNK_ROWS // 2
        HG = N_GROUPS // 2
        gA = gathers(b, 0, HR)
        gB = gathers(b, HR, CHUNK_ROWS)
        fire_edges(jnp.minimum(q + 1, N_CHUNKS - 1), nb)
        adj_dst(b)
        for g in gA:
            g.wait()
        multiply(b, 0, HG)
        sA = scatters(b, 0, HR)
        for g in gB:
            g.wait()
        multiply(b, HG, N_GROUPS)
        sB = scatters(b, HR, CHUNK_ROWS)
        wait_edges(nb)
        for sc in sA + sB:
            sc.wait()

    r0 = row0_of(0)
    pltpu.sync_copy(srcr.at[pl.ds(r0, CHUNK_ROWS)], src0)
    pltpu.sync_copy(dstr.at[pl.ds(r0, CHUNK_ROWS)], dst0)
    pltpu.sync_copy(wr.at[pl.ds(r0, CHUNK_ROWS)], w0)

    @pl.loop(0, N_CHUNKS // 2)
    def _(p):
        body(2 * p, 0)
        body(2 * p + 1, 1)

    plsc.subcore_barrier()
    pltpu.sync_copy(acc.at[pl.ds(s * READ_ROWS, READ_ROWS)],
                    out.at[pl.ds(c * HALF + s * READ_ROWS, READ_ROWS)])

    @pl.when(s == 0)
    def _():
        pltpu.sync_copy(acc.at[pl.ds(NS * READ_ROWS, READ_TAIL)],
                        out.at[pl.ds(c * HALF + NS * READ_ROWS, READ_TAIL)])


_layer = pl.kernel(
    _layer_body,
    out_type=jax.ShapeDtypeStruct((NN, D), jnp.float32),
    mesh=_mesh,
    scratch_types=[
        pltpu.VMEM_SHARED((ACC_ROWS, D), jnp.float32),
        pltpu.VMEM((CHUNK_ROWS, LANE), jnp.int32),
        pltpu.VMEM((CHUNK_ROWS, LANE), jnp.int32),
        pltpu.VMEM((CHUNK_ROWS, LANE), jnp.int32),
        pltpu.VMEM((CHUNK_ROWS, LANE), jnp.int32),
        pltpu.VMEM((CHUNK_ROWS, LANE), jnp.float32),
        pltpu.VMEM((CHUNK_ROWS, LANE), jnp.float32),
        pltpu.VMEM((CHUNK_E, D), jnp.float32),
        pltpu.SemaphoreType.DMA,
        pltpu.SemaphoreType.DMA,
        pltpu.SemaphoreType.DMA,
    ],
    compiler_params=pltpu.CompilerParams(use_tc_tiling_on_sc=False),
)

MEAN_CH = 624
MEAN_ROWS = 3120
MEAN_TAIL = NN - NC * NS * MEAN_ROWS


def _mean_body(x1, x2, x3, out, b1, b2, b3):
    c = lax.axis_index("c")
    s = lax.axis_index("s")
    wid = s * NC + c
    base = wid * MEAN_ROWS
    third = jnp.float32(1.0 / 3.0)

    def avg_rows(n_rows):
        @plsc.parallel_loop(0, n_rows * 2, 1, unroll=4)
        def _(t):
            r = t >> 1
            col = (t & 1) * 16
            v = (b1[r, pl.ds(col, 16)] + b2[r, pl.ds(col, 16)]
                 + b3[r, pl.ds(col, 16)]) * third
            b1[r, pl.ds(col, 16)] = v

    @pl.loop(0, MEAN_ROWS // MEAN_CH)
    def _(q):
        r0 = base + q * MEAN_CH
        pltpu.sync_copy(x1.at[pl.ds(r0, MEAN_CH)], b1)
        pltpu.sync_copy(x2.at[pl.ds(r0, MEAN_CH)], b2)
        pltpu.sync_copy(x3.at[pl.ds(r0, MEAN_CH)], b3)
        avg_rows(MEAN_CH)
        pltpu.sync_copy(b1, out.at[pl.ds(r0, MEAN_CH)])

    @pl.when(wid == 0)
    def _():
        t0 = NC * NS * MEAN_ROWS
        pltpu.sync_copy(x1.at[pl.ds(t0, MEAN_TAIL)], b1.at[pl.ds(0, MEAN_TAIL)])
        pltpu.sync_copy(x2.at[pl.ds(t0, MEAN_TAIL)], b2.at[pl.ds(0, MEAN_TAIL)])
        pltpu.sync_copy(x3.at[pl.ds(t0, MEAN_TAIL)], b3.at[pl.ds(0, MEAN_TAIL)])
        avg_rows(MEAN_TAIL)
        pltpu.sync_copy(b1.at[pl.ds(0, MEAN_TAIL)], out.at[pl.ds(t0, MEAN_TAIL)])


_mean = pl.kernel(
    _mean_body,
    out_type=jax.ShapeDtypeStruct((NN, D), jnp.float32),
    mesh=_mesh,
    scratch_types=[
        pltpu.VMEM((MEAN_CH, D), jnp.float32),
        pltpu.VMEM((MEAN_CH, D), jnp.float32),
        pltpu.VMEM((MEAN_CH, D), jnp.float32),
    ],
    compiler_params=pltpu.CompilerParams(use_tc_tiling_on_sc=False),
)


def kernel(user_emb, item_emb, edge_weight, edge_src, edge_dst):
    ego0 = jnp.concatenate([user_emb, item_emb], axis=0)

    pad = E_PAD - edge_src.shape[0]
    src = jnp.concatenate(
        [edge_src.astype(jnp.int32), jnp.zeros((pad,), jnp.int32)])
    dst = jnp.concatenate(
        [edge_dst.astype(jnp.int32), jnp.full((pad,), NN, jnp.int32)])
    w = jnp.concatenate([edge_weight, jnp.zeros((pad,), jnp.float32)])
    srcr = src.reshape(R_PAD, LANE)
    dstr = dst.reshape(R_PAD, LANE)
    wr = w.reshape(R_PAD, LANE)

    x1 = _layer(ego0, srcr, dstr, wr)
    x2 = _layer(x1, srcr, dstr, wr)
    x3 = _layer(x2, srcr, dstr, wr)
    final = _mean(x1, x2, x3)
    return (final[:USER_N], final[USER_N:])


# speedup vs baseline: 1.1411x; 1.1411x over previous
import jax
import jax.numpy as jnp
from jax import lax
from jax.experimental import pallas as pl
from jax.experimental.pallas import tpu as pltpu
from jax.experimental.pallas import tpu_sc as plsc

USER_N = 50000
ITEM_N = 50000
NN = USER_N + ITEM_N
D = 32
HALF = NN // 2

NC = 2
NS = 16

LANE = 128
CHUNK_ROWS = 3
ROWS_PER_TILE = 786
N_CHUNKS = ROWS_PER_TILE // CHUNK_ROWS
R_PAD = ROWS_PER_TILE * NS
E_PAD = R_PAD * LANE
CHUNK_E = CHUNK_ROWS * LANE
N_GROUPS = CHUNK_E // 16

ZCH = 224
ACC_ROWS = NS * 14 * ZCH
TRASH = HALF
READ_ROWS = 3120
READ_TAIL = HALF - NS * READ_ROWS

_mesh = plsc.VectorSubcoreMesh(core_axis_name="c", subcore_axis_name="s",
                               num_cores=NC, num_subcores=NS)


def _layer_body(ego, srcr, dstr, wr, out, acc,
                src0, src1, dst0, dst1, w0, w1, rows0, rows1,
                esem, gsem):
    c = lax.axis_index("c")
    s = lax.axis_index("s")
    zero16 = jnp.zeros((16,), jnp.float32)
    half_base = c * HALF

    srcb = (src0, src1)
    dstb = (dst0, dst1)
    wb = (w0, w1)
    rowsb = (rows0, rows1)

    @pl.loop(0, ZCH)
    def _(r):
        rows0[r, pl.ds(0, 16)] = zero16
        rows0[r, pl.ds(16, 16)] = zero16

    for q in range(14):
        pltpu.sync_copy(rows0.at[pl.ds(0, ZCH)],
                        acc.at[pl.ds((s * 14 + q) * ZCH, ZCH)])
    plsc.subcore_barrier()

    def row0_of(q):
        return s * ROWS_PER_TILE + q * CHUNK_ROWS

    def fire_edges(q, b):
        r0 = row0_of(q)
        pltpu.async_copy(srcr.at[pl.ds(r0, CHUNK_ROWS)], srcb[b], esem)
        pltpu.async_copy(dstr.at[pl.ds(r0, CHUNK_ROWS)], dstb[b], esem)
        pltpu.async_copy(wr.at[pl.ds(r0, CHUNK_ROWS)], wb[b], esem)

    def wait_edges(b):
        pltpu.make_async_copy(srcr.at[pl.ds(0, CHUNK_ROWS)], srcb[b], esem).wait()
        pltpu.make_async_copy(dstr.at[pl.ds(0, CHUNK_ROWS)], dstb[b], esem).wait()
        pltpu.make_async_copy(wr.at[pl.ds(0, CHUNK_ROWS)], wb[b], esem).wait()

    def gathers(b):
        return [pltpu.async_copy(ego.at[srcb[b].at[r]],
                                 rowsb[b].at[pl.ds(r * LANE, LANE)], gsem)
                for r in range(CHUNK_ROWS)]

    def sync_scatters(b):
        for r in range(CHUNK_ROWS):
            pltpu.sync_copy(rowsb[b].at[pl.ds(r * LANE, LANE)],
                            acc.at[dstb[b].at[r]], add=True)

    def adj_dst(b):
        d_ref = dstb[b]
        for j in range(CHUNK_ROWS):
            for i in range(LANE // 16):
                dv = d_ref[j, pl.ds(i * 16, 16)]
                lv = dv - half_base
                inr = (lv >= 0) & (lv < HALF)
                d_ref[j, pl.ds(i * 16, 16)] = jnp.where(inr, lv, TRASH)

    def multiply(b):
        rows = rowsb[b]
        w_v = wb[b]

        @plsc.parallel_loop(0, N_GROUPS, 1)
        def _(g):
            j = g >> 3
            i = (g & 7) * 16
            w16 = w_v[j, pl.ds(i, 16)]
            e0 = g * 16
            for base in range(0, 16, 8):
                vals = []
                for l in range(base, base + 8):
                    a = rows[e0 + l, pl.ds(0, 16)]
                    bb = rows[e0 + l, pl.ds(16, 16)]
                    vals.append((l, a, bb, w16[l]))
                for l, a, bb, w in vals:
                    rows[e0 + l, pl.ds(0, 16)] = a * w
                    rows[e0 + l, pl.ds(16, 16)] = bb * w

    def body(q, b):
        nb = 1 - b
        gs = gathers(b)
        adj_dst(b)
        for g in gs:
            g.wait()
        multiply(b)
        sync_scatters(b)
        wait_edges(nb)
        fire_edges(jnp.minimum(q + 2, N_CHUNKS - 1), b)

    r0 = row0_of(0)
    pltpu.sync_copy(srcr.at[pl.ds(r0, CHUNK_ROWS)], src0)
    pltpu.sync_copy(dstr.at[pl.ds(r0, CHUNK_ROWS)], dst0)
    pltpu.sync_copy(wr.at[pl.ds(r0, CHUNK_ROWS)], w0)
    fire_edges(jnp.int32(1), 1)

    @pl.loop(0, N_CHUNKS // 2)
    def _(p):
        body(2 * p, 0)
        body(2 * p + 1, 1)

    wait_edges(1)

    plsc.subcore_barrier()
    pltpu.sync_copy(acc.at[pl.ds(s * READ_ROWS, READ_ROWS)],
                    out.at[pl.ds(c * HALF + s * READ_ROWS, READ_ROWS)])

    @pl.when(s == 0)
    def _():
        pltpu.sync_copy(acc.at[pl.ds(NS * READ_ROWS, READ_TAIL)],
                        out.at[pl.ds(c * HALF + NS * READ_ROWS, READ_TAIL)])


_layer = pl.kernel(
    _layer_body,
    out_type=jax.ShapeDtypeStruct((NN, D), jnp.float32),
    mesh=_mesh,
    scratch_types=[
        pltpu.VMEM_SHARED((ACC_ROWS, D), jnp.float32),
        pltpu.VMEM((CHUNK_ROWS, LANE), jnp.int32),
        pltpu.VMEM((CHUNK_ROWS, LANE), jnp.int32),
        pltpu.VMEM((CHUNK_ROWS, LANE), jnp.int32),
        pltpu.VMEM((CHUNK_ROWS, LANE), jnp.int32),
        pltpu.VMEM((CHUNK_ROWS, LANE), jnp.float32),
        pltpu.VMEM((CHUNK_ROWS, LANE), jnp.float32),
        pltpu.VMEM((CHUNK_E, D), jnp.float32),
        pltpu.VMEM((CHUNK_E, D), jnp.float32),
        pltpu.SemaphoreType.DMA,
        pltpu.SemaphoreType.DMA,
    ],
    compiler_params=pltpu.CompilerParams(use_tc_tiling_on_sc=False),
)

MEAN_CH = 624
MEAN_ROWS = 3120
MEAN_TAIL = NN - NC * NS * MEAN_ROWS


def _mean_body(x1, x2, x3, out, b1, b2, b3):
    c = lax.axis_index("c")
    s = lax.axis_index("s")
    wid = s * NC + c
    base = wid * MEAN_ROWS
    third = jnp.float32(1.0 / 3.0)

    def avg_rows(n_rows):
        @plsc.parallel_loop(0, n_rows * 2, 1, unroll=4)
        def _(t):
            r = t >> 1
            col = (t & 1) * 16
            v = (b1[r, pl.ds(col, 16)] + b2[r, pl.ds(col, 16)]
                 + b3[r, pl.ds(col, 16)]) * third
            b1[r, pl.ds(col, 16)] = v

    @pl.loop(0, MEAN_ROWS // MEAN_CH)
    def _(q):
        r0 = base + q * MEAN_CH
        pltpu.sync_copy(x1.at[pl.ds(r0, MEAN_CH)], b1)
        pltpu.sync_copy(x2.at[pl.ds(r0, MEAN_CH)], b2)
        pltpu.sync_copy(x3.at[pl.ds(r0, MEAN_CH)], b3)
        avg_rows(MEAN_CH)
        pltpu.sync_copy(b1, out.at[pl.ds(r0, MEAN_CH)])

    @pl.when(wid == 0)
    def _():
        t0 = NC * NS * MEAN_ROWS
        pltpu.sync_copy(x1.at[pl.ds(t0, MEAN_TAIL)], b1.at[pl.ds(0, MEAN_TAIL)])
        pltpu.sync_copy(x2.at[pl.ds(t0, MEAN_TAIL)], b2.at[pl.ds(0, MEAN_TAIL)])
        pltpu.sync_copy(x3.at[pl.ds(t0, MEAN_TAIL)], b3.at[pl.ds(0, MEAN_TAIL)])
        avg_rows(MEAN_TAIL)
        pltpu.sync_copy(b1.at[pl.ds(0, MEAN_TAIL)], out.at[pl.ds(t0, MEAN_TAIL)])


_mean = pl.kernel(
    _mean_body,
    out_type=jax.ShapeDtypeStruct((NN, D), jnp.float32),
    mesh=_mesh,
    scratch_types=[
        pltpu.VMEM((MEAN_CH, D), jnp.float32),
        pltpu.VMEM((MEAN_CH, D), jnp.float32),
        pltpu.VMEM((MEAN_CH, D), jnp.float32),
    ],
    compiler_params=pltpu.CompilerParams(use_tc_tiling_on_sc=False),
)


def kernel(user_emb, item_emb, edge_weight, edge_src, edge_dst):
    ego0 = jnp.concatenate([user_emb, item_emb], axis=0)

    pad = E_PAD - edge_src.shape[0]
    src = jnp.concatenate(
        [edge_src.astype(jnp.int32), jnp.zeros((pad,), jnp.int32)])
    dst = jnp.concatenate(
        [edge_dst.astype(jnp.int32), jnp.full((pad,), NN, jnp.int32)])
    w = jnp.concatenate([edge_weight, jnp.zeros((pad,), jnp.float32)])
    srcr = src.reshape(R_PAD, LANE)
    dstr = dst.reshape(R_PAD, LANE)
    wr = w.reshape(R_PAD, LANE)

    x1 = _layer(ego0, srcr, dstr, wr)
    x2 = _layer(x1, srcr, dstr, wr)
    x3 = _layer(x2, srcr, dstr, wr)
    final = _mean(x1, x2, x3)
    return (final[:USER_N], final[USER_N:])


# speedup vs baseline: 1.6214x; 1.4209x over previous
import jax
import jax.numpy as jnp
from jax import lax
from jax.experimental import pallas as pl
from jax.experimental.pallas import tpu as pltpu
from jax.experimental.pallas import tpu_sc as plsc

USER_N = 50000
ITEM_N = 50000
NN = USER_N + ITEM_N
D = 32
HALF = NN // 2

NC = 2
NS = 16

LANE = 128
CHUNK_ROWS = 3
ROWS_PER_TILE = 786
N_CHUNKS = ROWS_PER_TILE // CHUNK_ROWS
R_PAD = ROWS_PER_TILE * NS
E_PAD = R_PAD * LANE
CHUNK_E = CHUNK_ROWS * LANE
N_GROUPS = CHUNK_E // 16

ZCH = 224
ACC_ROWS = NS * 14 * ZCH
TRASH = HALF
READ_ROWS = 3120
READ_TAIL = HALF - NS * READ_ROWS

_mesh = plsc.VectorSubcoreMesh(core_axis_name="c", subcore_axis_name="s",
                               num_cores=NC, num_subcores=NS)


def _layer_body(ego, srcr, dstr, wr, out, acc,
                src0, src1, dst0, dst1, w0, w1, rows0, rows1,
                esem, gsem):
    c = lax.axis_index("c")
    s = lax.axis_index("s")
    zero16 = jnp.zeros((16,), jnp.float32)
    half_base = c * HALF

    srcb = (src0, src1)
    dstb = (dst0, dst1)
    wb = (w0, w1)
    rowsb = (rows0, rows1)

    @pl.loop(0, ZCH)
    def _(r):
        rows0[r, pl.ds(0, 16)] = zero16
        rows0[r, pl.ds(16, 16)] = zero16

    for q in range(14):
        pltpu.sync_copy(rows0.at[pl.ds(0, ZCH)],
                        acc.at[pl.ds((s * 14 + q) * ZCH, ZCH)])
    plsc.subcore_barrier()

    def row0_of(q):
        return s * ROWS_PER_TILE + q * CHUNK_ROWS

    def fire_edges(q, b):
        r0 = row0_of(q)
        pltpu.async_copy(srcr.at[pl.ds(r0, CHUNK_ROWS)], srcb[b], esem)
        pltpu.async_copy(dstr.at[pl.ds(r0, CHUNK_ROWS)], dstb[b], esem)
        pltpu.async_copy(wr.at[pl.ds(r0, CHUNK_ROWS)], wb[b], esem)

    def wait_edges(b):
        pltpu.make_async_copy(srcr.at[pl.ds(0, CHUNK_ROWS)], srcb[b], esem).wait()
        pltpu.make_async_copy(dstr.at[pl.ds(0, CHUNK_ROWS)], dstb[b], esem).wait()
        pltpu.make_async_copy(wr.at[pl.ds(0, CHUNK_ROWS)], wb[b], esem).wait()

    def gathers(b):
        return [pltpu.async_copy(ego.at[srcb[b].at[r]],
                                 rowsb[b].at[pl.ds(r * LANE, LANE)], gsem)
                for r in range(CHUNK_ROWS)]

    def sync_scatters(b):
        for r in range(CHUNK_ROWS):
            pltpu.sync_copy(
                rowsb[b].at[pl.ds(r * LANE, LANE)],
                acc.at[plsc.Indices(dstb[b].at[r], ignored_value=TRASH)],
                add=True)

    def adj_dst(b):
        d_ref = dstb[b]
        for j in range(CHUNK_ROWS):
            for i in range(LANE // 16):
                dv = d_ref[j, pl.ds(i * 16, 16)]
                lv = dv - half_base
                inr = (lv >= 0) & (lv < HALF)
                d_ref[j, pl.ds(i * 16, 16)] = jnp.where(inr, lv, TRASH)

    def multiply(b):
        rows = rowsb[b]
        w_v = wb[b]

        @plsc.parallel_loop(0, N_GROUPS, 1)
        def _(g):
            j = g >> 3
            i = (g & 7) * 16
            w16 = w_v[j, pl.ds(i, 16)]
            e0 = g * 16
            for base in range(0, 16, 8):
                vals = []
                for l in range(base, base + 8):
                    a = rows[e0 + l, pl.ds(0, 16)]
                    bb = rows[e0 + l, pl.ds(16, 16)]
                    vals.append((l, a, bb, w16[l]))
                for l, a, bb, w in vals:
                    rows[e0 + l, pl.ds(0, 16)] = a * w
                    rows[e0 + l, pl.ds(16, 16)] = bb * w

    def body(q, b):
        nb = 1 - b
        gs = gathers(b)
        adj_dst(b)
        for g in gs:
            g.wait()
        multiply(b)
        sync_scatters(b)
        wait_edges(nb)
        fire_edges(jnp.minimum(q + 2, N_CHUNKS - 1), b)

    r0 = row0_of(0)
    pltpu.sync_copy(srcr.at[pl.ds(r0, CHUNK_ROWS)], src0)
    pltpu.sync_copy(dstr.at[pl.ds(r0, CHUNK_ROWS)], dst0)
    pltpu.sync_copy(wr.at[pl.ds(r0, CHUNK_ROWS)], w0)
    fire_edges(jnp.int32(1), 1)

    @pl.loop(0, N_CHUNKS // 2)
    def _(p):
        body(2 * p, 0)
        body(2 * p + 1, 1)

    wait_edges(1)

    plsc.subcore_barrier()
    pltpu.sync_copy(acc.at[pl.ds(s * READ_ROWS, READ_ROWS)],
                    out.at[pl.ds(c * HALF + s * READ_ROWS, READ_ROWS)])

    @pl.when(s == 0)
    def _():
        pltpu.sync_copy(acc.at[pl.ds(NS * READ_ROWS, READ_TAIL)],
                        out.at[pl.ds(c * HALF + NS * READ_ROWS, READ_TAIL)])


_layer = pl.kernel(
    _layer_body,
    out_type=jax.ShapeDtypeStruct((NN, D), jnp.float32),
    mesh=_mesh,
    scratch_types=[
        pltpu.VMEM_SHARED((ACC_ROWS, D), jnp.float32),
        pltpu.VMEM((CHUNK_ROWS, LANE), jnp.int32),
        pltpu.VMEM((CHUNK_ROWS, LANE), jnp.int32),
        pltpu.VMEM((CHUNK_ROWS, LANE), jnp.int32),
        pltpu.VMEM((CHUNK_ROWS, LANE), jnp.int32),
        pltpu.VMEM((CHUNK_ROWS, LANE), jnp.float32),
        pltpu.VMEM((CHUNK_ROWS, LANE), jnp.float32),
        pltpu.VMEM((CHUNK_E, D), jnp.float32),
        pltpu.VMEM((CHUNK_E, D), jnp.float32),
        pltpu.SemaphoreType.DMA,
        pltpu.SemaphoreType.DMA,
    ],
    compiler_params=pltpu.CompilerParams(use_tc_tiling_on_sc=False),
)

MEAN_CH = 624
MEAN_ROWS = 3120
MEAN_TAIL = NN - NC * NS * MEAN_ROWS


def _mean_body(x1, x2, x3, out, b1, b2, b3):
    c = lax.axis_index("c")
    s = lax.axis_index("s")
    wid = s * NC + c
    base = wid * MEAN_ROWS
    third = jnp.float32(1.0 / 3.0)

    def avg_rows(n_rows):
        @plsc.parallel_loop(0, n_rows * 2, 1, unroll=4)
        def _(t):
            r = t >> 1
            col = (t & 1) * 16
            v = (b1[r, pl.ds(col, 16)] + b2[r, pl.ds(col, 16)]
                 + b3[r, pl.ds(col, 16)]) * third
            b1[r, pl.ds(col, 16)] = v

    @pl.loop(0, MEAN_ROWS // MEAN_CH)
    def _(q):
        r0 = base + q * MEAN_CH
        pltpu.sync_copy(x1.at[pl.ds(r0, MEAN_CH)], b1)
        pltpu.sync_copy(x2.at[pl.ds(r0, MEAN_CH)], b2)
        pltpu.sync_copy(x3.at[pl.ds(r0, MEAN_CH)], b3)
        avg_rows(MEAN_CH)
        pltpu.sync_copy(b1, out.at[pl.ds(r0, MEAN_CH)])

    @pl.when(wid == 0)
    def _():
        t0 = NC * NS * MEAN_ROWS
        pltpu.sync_copy(x1.at[pl.ds(t0, MEAN_TAIL)], b1.at[pl.ds(0, MEAN_TAIL)])
        pltpu.sync_copy(x2.at[pl.ds(t0, MEAN_TAIL)], b2.at[pl.ds(0, MEAN_TAIL)])
        pltpu.sync_copy(x3.at[pl.ds(t0, MEAN_TAIL)], b3.at[pl.ds(0, MEAN_TAIL)])
        avg_rows(MEAN_TAIL)
        pltpu.sync_copy(b1.at[pl.ds(0, MEAN_TAIL)], out.at[pl.ds(t0, MEAN_TAIL)])


_mean = pl.kernel(
    _mean_body,
    out_type=jax.ShapeDtypeStruct((NN, D), jnp.float32),
    mesh=_mesh,
    scratch_types=[
        pltpu.VMEM((MEAN_CH, D), jnp.float32),
        pltpu.VMEM((MEAN_CH, D), jnp.float32),
        pltpu.VMEM((MEAN_CH, D), jnp.float32),
    ],
    compiler_params=pltpu.CompilerParams(use_tc_tiling_on_sc=False),
)


def kernel(user_emb, item_emb, edge_weight, edge_src, edge_dst):
    ego0 = jnp.concatenate([user_emb, item_emb], axis=0)

    pad = E_PAD - edge_src.shape[0]
    src = jnp.concatenate(
        [edge_src.astype(jnp.int32), jnp.zeros((pad,), jnp.int32)])
    dst = jnp.concatenate(
        [edge_dst.astype(jnp.int32), jnp.full((pad,), NN, jnp.int32)])
    w = jnp.concatenate([edge_weight, jnp.zeros((pad,), jnp.float32)])
    srcr = src.reshape(R_PAD, LANE)
    dstr = dst.reshape(R_PAD, LANE)
    wr = w.reshape(R_PAD, LANE)

    x1 = _layer(ego0, srcr, dstr, wr)
    x2 = _layer(x1, srcr, dstr, wr)
    x3 = _layer(x2, srcr, dstr, wr)
    final = _mean(x1, x2, x3)
    return (final[:USER_N], final[USER_N:])


# speedup vs baseline: 2.0128x; 1.2414x over previous
import jax
import jax.numpy as jnp
from jax import lax
from jax.experimental import pallas as pl
from jax.experimental.pallas import tpu as pltpu
from jax.experimental.pallas import tpu_sc as plsc

USER_N = 50000
ITEM_N = 50000
NN = USER_N + ITEM_N
D = 32
HALF = NN // 2

NC = 2
NS = 16

LANE = 128
CHUNK_ROWS = 3
ROWS_PER_TILE = 786
N_CHUNKS = ROWS_PER_TILE // CHUNK_ROWS
R_PAD = ROWS_PER_TILE * NS
E_PAD = R_PAD * LANE
CHUNK_E = CHUNK_ROWS * LANE
N_GROUPS = CHUNK_E // 16

ZCH = 224
ACC_ROWS = NS * 14 * ZCH
TRASH = HALF
READ_ROWS = 3120
READ_TAIL = HALF - NS * READ_ROWS

_mesh = plsc.VectorSubcoreMesh(core_axis_name="c", subcore_axis_name="s",
                               num_cores=NC, num_subcores=NS)


def _layer_body(ego, srcr, dstr, wr, out, acc,
                src0, src1, dst0, dst1, w0, w1, g0, g1, rows0, rows1,
                esem, gsem):
    c = lax.axis_index("c")
    s = lax.axis_index("s")
    zero16 = jnp.zeros((16,), jnp.float32)
    half_base = c * HALF

    srcb = (src0, src1)
    dstb = (dst0, dst1)
    wb = (w0, w1)
    gb = (g0, g1)
    rowsb = (rows0, rows1)

    @pl.loop(0, ZCH)
    def _(r):
        rows0[r, pl.ds(0, 16)] = zero16
        rows0[r, pl.ds(16, 16)] = zero16

    for q in range(14):
        pltpu.sync_copy(rows0.at[pl.ds(0, ZCH)],
                        acc.at[pl.ds((s * 14 + q) * ZCH, ZCH)])
    plsc.subcore_barrier()

    def row0_of(q):
        return s * ROWS_PER_TILE + q * CHUNK_ROWS

    def fire_edges(q, b):
        r0 = row0_of(q)
        pltpu.async_copy(srcr.at[pl.ds(r0, CHUNK_ROWS)], srcb[b], esem)
        pltpu.async_copy(dstr.at[pl.ds(r0, CHUNK_ROWS)], dstb[b], esem)
        pltpu.async_copy(wr.at[pl.ds(r0, CHUNK_ROWS)], wb[b], esem)

    def wait_edges(b):
        pltpu.make_async_copy(srcr.at[pl.ds(0, CHUNK_ROWS)], srcb[b], esem).wait()
        pltpu.make_async_copy(dstr.at[pl.ds(0, CHUNK_ROWS)], dstb[b], esem).wait()
        pltpu.make_async_copy(wr.at[pl.ds(0, CHUNK_ROWS)], wb[b], esem).wait()

    def gathers(b):
        return [pltpu.async_copy(
                    ego.at[plsc.Indices(gb[b].at[r], ignored_value=NN)],
                    rowsb[b].at[pl.ds(r * LANE, LANE)], gsem)
                for r in range(CHUNK_ROWS)]

    def sync_scatters(b):
        for r in range(CHUNK_ROWS):
            pltpu.sync_copy(
                rowsb[b].at[pl.ds(r * LANE, LANE)],
                acc.at[plsc.Indices(dstb[b].at[r], ignored_value=TRASH)],
                add=True)

    def adj_dst(b):
        d_ref = dstb[b]
        s_ref = srcb[b]
        g_ref = gb[b]
        for j in range(CHUNK_ROWS):
            for i in range(LANE // 16):
                dv = d_ref[j, pl.ds(i * 16, 16)]
                sv = s_ref[j, pl.ds(i * 16, 16)]
                lv = dv - half_base
                inr = (lv >= 0) & (lv < HALF)
                d_ref[j, pl.ds(i * 16, 16)] = jnp.where(inr, lv, TRASH)
                g_ref[j, pl.ds(i * 16, 16)] = jnp.where(inr, sv, NN)

    def multiply(b):
        rows = rowsb[b]
        w_v = wb[b]

        @plsc.parallel_loop(0, N_GROUPS, 1)
        def _(g):
            j = g >> 3
            i = (g & 7) * 16
            w16 = w_v[j, pl.ds(i, 16)]
            e0 = g * 16
            for base in range(0, 16, 8):
                vals = []
                for l in range(base, base + 8):
                    a = rows[e0 + l, pl.ds(0, 16)]
                    bb = rows[e0 + l, pl.ds(16, 16)]
                    vals.append((l, a, bb, w16[l]))
                for l, a, bb, w in vals:
                    rows[e0 + l, pl.ds(0, 16)] = a * w
                    rows[e0 + l, pl.ds(16, 16)] = bb * w

    def body(q, b):
        nb = 1 - b
        gs = gathers(b)
        wait_edges(nb)
        adj_dst(nb)
        for g in gs:
            g.wait()
        multiply(b)
        sync_scatters(b)
        fire_edges(jnp.minimum(q + 2, N_CHUNKS - 1), b)

    r0 = row0_of(0)
    pltpu.sync_copy(srcr.at[pl.ds(r0, CHUNK_ROWS)], src0)
    pltpu.sync_copy(dstr.at[pl.ds(r0, CHUNK_ROWS)], dst0)
    pltpu.sync_copy(wr.at[pl.ds(r0, CHUNK_ROWS)], w0)
    adj_dst(0)
    fire_edges(jnp.int32(1), 1)

    @pl.loop(0, N_CHUNKS // 2)
    def _(p):
        body(2 * p, 0)
        body(2 * p + 1, 1)

    wait_edges(1)

    plsc.subcore_barrier()
    pltpu.sync_copy(acc.at[pl.ds(s * READ_ROWS, READ_ROWS)],
                    out.at[pl.ds(c * HALF + s * READ_ROWS, READ_ROWS)])

    @pl.when(s == 0)
    def _():
        pltpu.sync_copy(acc.at[pl.ds(NS * READ_ROWS, READ_TAIL)],
                        out.at[pl.ds(c * HALF + NS * READ_ROWS, READ_TAIL)])


_layer = pl.kernel(
    _layer_body,
    out_type=jax.ShapeDtypeStruct((NN, D), jnp.float32),
    mesh=_mesh,
    scratch_types=[
        pltpu.VMEM_SHARED((ACC_ROWS, D), jnp.float32),
        pltpu.VMEM((CHUNK_ROWS, LANE), jnp.int32),
        pltpu.VMEM((CHUNK_ROWS, LANE), jnp.int32),
        pltpu.VMEM((CHUNK_ROWS, LANE), jnp.int32),
        pltpu.VMEM((CHUNK_ROWS, LANE), jnp.int32),
        pltpu.VMEM((CHUNK_ROWS, LANE), jnp.float32),
        pltpu.VMEM((CHUNK_ROWS, LANE), jnp.float32),
        pltpu.VMEM((CHUNK_ROWS, LANE), jnp.int32),
        pltpu.VMEM((CHUNK_ROWS, LANE), jnp.int32),
        pltpu.VMEM((CHUNK_E, D), jnp.float32),
        pltpu.VMEM((CHUNK_E, D), jnp.float32),
        pltpu.SemaphoreType.DMA,
        pltpu.SemaphoreType.DMA,
    ],
    compiler_params=pltpu.CompilerParams(use_tc_tiling_on_sc=False),
)

MEAN_CH = 624
MEAN_ROWS = 3120
MEAN_TAIL = NN - NC * NS * MEAN_ROWS


def _mean_body(x1, x2, x3, out, b1, b2, b3):
    c = lax.axis_index("c")
    s = lax.axis_index("s")
    wid = s * NC + c
    base = wid * MEAN_ROWS
    third = jnp.float32(1.0 / 3.0)

    def avg_rows(n_rows):
        @plsc.parallel_loop(0, n_rows * 2, 1, unroll=4)
        def _(t):
            r = t >> 1
            col = (t & 1) * 16
            v = (b1[r, pl.ds(col, 16)] + b2[r, pl.ds(col, 16)]
                 + b3[r, pl.ds(col, 16)]) * third
            b1[r, pl.ds(col, 16)] = v

    @pl.loop(0, MEAN_ROWS // MEAN_CH)
    def _(q):
        r0 = base + q * MEAN_CH
        pltpu.sync_copy(x1.at[pl.ds(r0, MEAN_CH)], b1)
        pltpu.sync_copy(x2.at[pl.ds(r0, MEAN_CH)], b2)
        pltpu.sync_copy(x3.at[pl.ds(r0, MEAN_CH)], b3)
        avg_rows(MEAN_CH)
        pltpu.sync_copy(b1, out.at[pl.ds(r0, MEAN_CH)])

    @pl.when(wid == 0)
    def _():
        t0 = NC * NS * MEAN_ROWS
        pltpu.sync_copy(x1.at[pl.ds(t0, MEAN_TAIL)], b1.at[pl.ds(0, MEAN_TAIL)])
        pltpu.sync_copy(x2.at[pl.ds(t0, MEAN_TAIL)], b2.at[pl.ds(0, MEAN_TAIL)])
        pltpu.sync_copy(x3.at[pl.ds(t0, MEAN_TAIL)], b3.at[pl.ds(0, MEAN_TAIL)])
        avg_rows(MEAN_TAIL)
        pltpu.sync_copy(b1.at[pl.ds(0, MEAN_TAIL)], out.at[pl.ds(t0, MEAN_TAIL)])


_mean = pl.kernel(
    _mean_body,
    out_type=jax.ShapeDtypeStruct((NN, D), jnp.float32),
    mesh=_mesh,
    scratch_types=[
        pltpu.VMEM((MEAN_CH, D), jnp.float32),
        pltpu.VMEM((MEAN_CH, D), jnp.float32),
        pltpu.VMEM((MEAN_CH, D), jnp.float32),
    ],
    compiler_params=pltpu.CompilerParams(use_tc_tiling_on_sc=False),
)


def kernel(user_emb, item_emb, edge_weight, edge_src, edge_dst):
    ego0 = jnp.concatenate([user_emb, item_emb], axis=0)

    pad = E_PAD - edge_src.shape[0]
    src = jnp.concatenate(
        [edge_src.astype(jnp.int32), jnp.zeros((pad,), jnp.int32)])
    dst = jnp.concatenate(
        [edge_dst.astype(jnp.int32), jnp.full((pad,), NN, jnp.int32)])
    w = jnp.concatenate([edge_weight, jnp.zeros((pad,), jnp.float32)])
    srcr = src.reshape(R_PAD, LANE)
    dstr = dst.reshape(R_PAD, LANE)
    wr = w.reshape(R_PAD, LANE)

    x1 = _layer(ego0, srcr, dstr, wr)
    x2 = _layer(x1, srcr, dstr, wr)
    x3 = _layer(x2, srcr, dstr, wr)
    final = _mean(x1, x2, x3)
    return (final[:USER_N], final[USER_N:])


# speedup vs baseline: 2.1156x; 1.0511x over previous
import jax
import jax.numpy as jnp
from jax import lax
from jax.experimental import pallas as pl
from jax.experimental.pallas import tpu as pltpu
from jax.experimental.pallas import tpu_sc as plsc

USER_N = 50000
ITEM_N = 50000
NN = USER_N + ITEM_N
D = 32
HALF = NN // 2

NC = 2
NS = 16

LANE = 128
CHUNK_ROWS = 3
ROWS_PER_TILE = 786
N_CHUNKS = ROWS_PER_TILE // CHUNK_ROWS
R_PAD = ROWS_PER_TILE * NS
E_PAD = R_PAD * LANE
CHUNK_E = CHUNK_ROWS * LANE
N_GROUPS = CHUNK_E // 16

ZCH = 224
ACC_ROWS = NS * 14 * ZCH
TRASH = HALF
READ_ROWS = 3120
READ_TAIL = HALF - NS * READ_ROWS

_mesh = plsc.VectorSubcoreMesh(core_axis_name="c", subcore_axis_name="s",
                               num_cores=NC, num_subcores=NS)


def _layer_body(ego, srcr, dstr, wr, out, acc,
                src0, src1, dst0, dst1, w0, w1, g0, g1, rows0, rows1,
                esem, gsem):
    c = lax.axis_index("c")
    s = lax.axis_index("s")
    zero16 = jnp.zeros((16,), jnp.float32)
    half_base = c * HALF

    srcb = (src0, src1)
    dstb = (dst0, dst1)
    wb = (w0, w1)
    gb = (g0, g1)
    rowsb = (rows0, rows1)

    @pl.loop(0, ZCH)
    def _(r):
        rows0[r, pl.ds(0, 16)] = zero16
        rows0[r, pl.ds(16, 16)] = zero16

    for q in range(14):
        pltpu.sync_copy(rows0.at[pl.ds(0, ZCH)],
                        acc.at[pl.ds((s * 14 + q) * ZCH, ZCH)])
    plsc.subcore_barrier()

    def row0_of(q):
        return s * ROWS_PER_TILE + q * CHUNK_ROWS

    def fire_edges(q, b):
        r0 = row0_of(q)
        pltpu.async_copy(srcr.at[pl.ds(r0, CHUNK_ROWS)], srcb[b], esem)
        pltpu.async_copy(dstr.at[pl.ds(r0, CHUNK_ROWS)], dstb[b], esem)
        pltpu.async_copy(wr.at[pl.ds(r0, CHUNK_ROWS)], wb[b], esem)

    def wait_edges(b):
        pltpu.make_async_copy(srcr.at[pl.ds(0, CHUNK_ROWS)], srcb[b], esem).wait()
        pltpu.make_async_copy(dstr.at[pl.ds(0, CHUNK_ROWS)], dstb[b], esem).wait()
        pltpu.make_async_copy(wr.at[pl.ds(0, CHUNK_ROWS)], wb[b], esem).wait()

    def gathers(b):
        return [pltpu.async_copy(
                    ego.at[plsc.Indices(gb[b].at[r], ignored_value=NN)],
                    rowsb[b].at[pl.ds(r * LANE, LANE)], gsem)
                for r in range(CHUNK_ROWS)]

    def sync_scatters(b):
        cps = [pltpu.async_copy(
                   rowsb[b].at[pl.ds(r * LANE, LANE)],
                   acc.at[plsc.Indices(dstb[b].at[r], ignored_value=TRASH)],
                   gsem, add=True)
               for r in range(CHUNK_ROWS)]
        for cp in cps:
            cp.wait()

    def adj_dst(b):
        d_ref = dstb[b]
        s_ref = srcb[b]
        g_ref = gb[b]
        for j in range(CHUNK_ROWS):
            for i in range(LANE // 16):
                dv = d_ref[j, pl.ds(i * 16, 16)]
                sv = s_ref[j, pl.ds(i * 16, 16)]
                lv = dv - half_base
                inr = (lv >= 0) & (lv < HALF)
                d_ref[j, pl.ds(i * 16, 16)] = jnp.where(inr, lv, TRASH)
                g_ref[j, pl.ds(i * 16, 16)] = jnp.where(inr, sv, NN)

    def multiply(b):
        rows = rowsb[b]
        w_v = wb[b]

        @plsc.parallel_loop(0, N_GROUPS, 1)
        def _(g):
            j = g >> 3
            i = (g & 7) * 16
            w16 = w_v[j, pl.ds(i, 16)]
            e0 = g * 16
            for base in range(0, 16, 8):
                vals = []
                for l in range(base, base + 8):
                    a = rows[e0 + l, pl.ds(0, 16)]
                    bb = rows[e0 + l, pl.ds(16, 16)]
                    vals.append((l, a, bb, w16[l]))
                for l, a, bb, w in vals:
                    rows[e0 + l, pl.ds(0, 16)] = a * w
                    rows[e0 + l, pl.ds(16, 16)] = bb * w

    def body(q, b):
        nb = 1 - b
        gs = gathers(b)
        wait_edges(nb)
        adj_dst(nb)
        for g in gs:
            g.wait()
        multiply(b)
        sync_scatters(b)
        fire_edges(jnp.minimum(q + 2, N_CHUNKS - 1), b)

    r0 = row0_of(0)
    pltpu.sync_copy(srcr.at[pl.ds(r0, CHUNK_ROWS)], src0)
    pltpu.sync_copy(dstr.at[pl.ds(r0, CHUNK_ROWS)], dst0)
    pltpu.sync_copy(wr.at[pl.ds(r0, CHUNK_ROWS)], w0)
    adj_dst(0)
    fire_edges(jnp.int32(1), 1)

    @pl.loop(0, N_CHUNKS // 2)
    def _(p):
        body(2 * p, 0)
        body(2 * p + 1, 1)

    wait_edges(1)

    plsc.subcore_barrier()
    pltpu.sync_copy(acc.at[pl.ds(s * READ_ROWS, READ_ROWS)],
                    out.at[pl.ds(c * HALF + s * READ_ROWS, READ_ROWS)])

    @pl.when(s == 0)
    def _():
        pltpu.sync_copy(acc.at[pl.ds(NS * READ_ROWS, READ_TAIL)],
                        out.at[pl.ds(c * HALF + NS * READ_ROWS, READ_TAIL)])


_layer = pl.kernel(
    _layer_body,
    out_type=jax.ShapeDtypeStruct((NN, D), jnp.float32),
    mesh=_mesh,
    scratch_types=[
        pltpu.VMEM_SHARED((ACC_ROWS, D), jnp.float32),
        pltpu.VMEM((CHUNK_ROWS, LANE), jnp.int32),
        pltpu.VMEM((CHUNK_ROWS, LANE), jnp.int32),
        pltpu.VMEM((CHUNK_ROWS, LANE), jnp.int32),
        pltpu.VMEM((CHUNK_ROWS, LANE), jnp.int32),
        pltpu.VMEM((CHUNK_ROWS, LANE), jnp.float32),
        pltpu.VMEM((CHUNK_ROWS, LANE), jnp.float32),
        pltpu.VMEM((CHUNK_ROWS, LANE), jnp.int32),
        pltpu.VMEM((CHUNK_ROWS, LANE), jnp.int32),
        pltpu.VMEM((CHUNK_E, D), jnp.float32),
        pltpu.VMEM((CHUNK_E, D), jnp.float32),
        pltpu.SemaphoreType.DMA,
        pltpu.SemaphoreType.DMA,
    ],
    compiler_params=pltpu.CompilerParams(use_tc_tiling_on_sc=False),
)

MEAN_CH = 624
MEAN_ROWS = 3120
MEAN_TAIL = NN - NC * NS * MEAN_ROWS


def _mean_body(x1, x2, x3, out, b1, b2, b3):
    c = lax.axis_index("c")
    s = lax.axis_index("s")
    wid = s * NC + c
    base = wid * MEAN_ROWS
    third = jnp.float32(1.0 / 3.0)

    def avg_rows(n_rows):
        @plsc.parallel_loop(0, n_rows * 2, 1, unroll=4)
        def _(t):
            r = t >> 1
            col = (t & 1) * 16
            v = (b1[r, pl.ds(col, 16)] + b2[r, pl.ds(col, 16)]
                 + b3[r, pl.ds(col, 16)]) * third
            b1[r, pl.ds(col, 16)] = v

    @pl.loop(0, MEAN_ROWS // MEAN_CH)
    def _(q):
        r0 = base + q * MEAN_CH
        pltpu.sync_copy(x1.at[pl.ds(r0, MEAN_CH)], b1)
        pltpu.sync_copy(x2.at[pl.ds(r0, MEAN_CH)], b2)
        pltpu.sync_copy(x3.at[pl.ds(r0, MEAN_CH)], b3)
        avg_rows(MEAN_CH)
        pltpu.sync_copy(b1, out.at[pl.ds(r0, MEAN_CH)])

    @pl.when(wid == 0)
    def _():
        t0 = NC * NS * MEAN_ROWS
        pltpu.sync_copy(x1.at[pl.ds(t0, MEAN_TAIL)], b1.at[pl.ds(0, MEAN_TAIL)])
        pltpu.sync_copy(x2.at[pl.ds(t0, MEAN_TAIL)], b2.at[pl.ds(0, MEAN_TAIL)])
        pltpu.sync_copy(x3.at[pl.ds(t0, MEAN_TAIL)], b3.at[pl.ds(0, MEAN_TAIL)])
        avg_rows(MEAN_TAIL)
        pltpu.sync_copy(b1.at[pl.ds(0, MEAN_TAIL)], out.at[pl.ds(t0, MEAN_TAIL)])


_mean = pl.kernel(
    _mean_body,
    out_type=jax.ShapeDtypeStruct((NN, D), jnp.float32),
    mesh=_mesh,
    scratch_types=[
        pltpu.VMEM((MEAN_CH, D), jnp.float32),
        pltpu.VMEM((MEAN_CH, D), jnp.float32),
        pltpu.VMEM((MEAN_CH, D), jnp.float32),
    ],
    compiler_params=pltpu.CompilerParams(use_tc_tiling_on_sc=False),
)


def kernel(user_emb, item_emb, edge_weight, edge_src, edge_dst):
    ego0 = jnp.concatenate([user_emb, item_emb], axis=0)

    pad = E_PAD - edge_src.shape[0]
    src = jnp.concatenate(
        [edge_src.astype(jnp.int32), jnp.zeros((pad,), jnp.int32)])
    dst = jnp.concatenate(
        [edge_dst.astype(jnp.int32), jnp.full((pad,), NN, jnp.int32)])
    w = jnp.concatenate([edge_weight, jnp.zeros((pad,), jnp.float32)])
    srcr = src.reshape(R_PAD, LANE)
    dstr = dst.reshape(R_PAD, LANE)
    wr = w.reshape(R_PAD, LANE)

    x1 = _layer(ego0, srcr, dstr, wr)
    x2 = _layer(x1, srcr, dstr, wr)
    x3 = _layer(x2, srcr, dstr, wr)
    final = _mean(x1, x2, x3)
    return (final[:USER_N], final[USER_N:])
